# Initial kernel scaffold; baseline (speedup 1.0000x reference)
import numpy as np
import jax
import jax.numpy as jnp
from jax.experimental import pallas as pl
from jax.experimental.pallas import tpu as pltpu

POS_T = 0.7
NEG_T = 0.3
MAX_POS = 128
MAX_NEG = 256
A = 20000
B = 8
G = 64
AP = 20480
ROWS = AP // 128
PAD_RANK = 1 << 20

_rank_cache = []


def _rank_tables():
    if not _rank_cache:
        cpu = jax.devices("cpu")[0]
        rkp = np.full((B, AP), PAD_RANK, np.int32)
        rkn = np.full((B, AP), PAD_RANK, np.int32)
        with jax.default_device(cpu):
            key = jax.random.key(42)
            for b in range(B):
                kp, kn = jax.random.split(jax.random.fold_in(key, b))
                for arr, kk in ((rkp, kp), (rkn, kn)):
                    r = np.asarray(jax.random.uniform(kk, (A,)))
                    order = np.argsort(-r, kind="stable")
                    rank = np.empty(A, np.int32)
                    rank[order] = np.arange(A, dtype=np.int32)
                    arr[b, :A] = rank
        _rank_cache.append((rkp.reshape(B, ROWS, 128), rkn.reshape(B, ROWS, 128)))
    return _rank_cache[0]


BLK = 16


def _iou_kernel(aT_ref, gt_ref, lab_ref, pos_ref, neg_ref,
                tx_ref, ty_ref, tw_ref, th_ref):
    ax1 = aT_ref[0]
    ay1 = aT_ref[1]
    ax2 = aT_ref[2]
    ay2 = aT_ref[3]
    aw = ax2 - ax1
    ah = ay2 - ay1
    acx = (ax1 + ax2) / 2.0
    acy = (ay1 + ay2) / 2.0
    area1 = jnp.maximum(aw, 0.0) * jnp.maximum(ah, 0.0)

    shape = ax1.shape
    best_i = jnp.full(shape, -1.0, jnp.float32)
    best_u = jnp.full(shape, 1.0, jnp.float32)
    bbx = jnp.zeros(shape, jnp.float32)
    bby = jnp.zeros(shape, jnp.float32)
    bbw = jnp.ones(shape, jnp.float32)
    bbh = jnp.ones(shape, jnp.float32)
    pos_any = jnp.zeros(shape, jnp.bool_)
    negv_any = jnp.zeros(shape, jnp.bool_)

    for j in range(G):
        gx1 = gt_ref[0, 0, j]
        gy1 = gt_ref[0, 1, j]
        gx2 = gt_ref[0, 2, j]
        gy2 = gt_ref[0, 3, j]
        valid = lab_ref[0, j] >= 0
        area2 = jnp.maximum(gx2 - gx1, 0.0) * jnp.maximum(gy2 - gy1, 0.0)

        ix1 = jnp.maximum(ax1, gx1)
        iy1 = jnp.maximum(ay1, gy1)
        ix2 = jnp.minimum(ax2, gx2)
        iy2 = jnp.minimum(ay2, gy2)
        iw = jnp.maximum(ix2 - ix1, 0.0)
        ih = jnp.maximum(iy2 - iy1, 0.0)
        inter = iw * ih
        s = area1 + area2
        pos_any = jnp.logical_or(pos_any,
                                 jnp.logical_and(17.0 * inter >= 7.0 * s, valid))
        negv_any = jnp.logical_or(negv_any,
                                  jnp.logical_and(13.0 * inter > 3.0 * s, valid))
        u = s - inter
        better = jnp.logical_and(inter * best_u > best_i * u, valid)
        best_i = jnp.where(better, inter, best_i)
        best_u = jnp.where(better, u, best_u)
        bbx = jnp.where(better, (gx1 + gx2) / 2.0, bbx)
        bby = jnp.where(better, (gy1 + gy2) / 2.0, bby)
        bbw = jnp.where(better, gx2 - gx1, bbw)
        bbh = jnp.where(better, gy2 - gy1, bbh)

    blk = pl.program_id(1)
    row = jax.lax.broadcasted_iota(jnp.int32, shape, 0)
    col = jax.lax.broadcasted_iota(jnp.int32, shape, 1)
    idx = (blk * BLK + row) * 128 + col
    in_range = idx < A
    pos_ref[0] = jnp.logical_and(pos_any, in_range).astype(jnp.int32)
    neg_ref[0] = jnp.logical_and(jnp.logical_not(negv_any), in_range).astype(jnp.int32)
    tx_ref[0] = (bbx - acx) / aw
    ty_ref[0] = (bby - acy) / ah
    tw_ref[0] = jnp.log(bbw / aw)
    th_ref[0] = jnp.log(bbh / ah)


def _thresh(mask, rank, k):
    t = jnp.int32(0)
    for b in range(14, -1, -1):
        cand = t + (1 << b)
        cnt = jnp.sum(jnp.where(jnp.logical_and(mask, rank < cand), 1, 0))
        t = jnp.where(cnt < k, cand, t)
    return t + 1


def _select_kernel(pos_ref, neg_ref, rkp_ref, rkn_ref,
                   tx_ref, ty_ref, tw_ref, th_ref, reg_ref, cls_ref):
    pos = pos_ref[0] != 0
    neg = neg_ref[0] != 0
    rkp = rkp_ref[0]
    rkn = rkn_ref[0]
    t_pos = _thresh(pos, rkp, MAX_POS)
    t_neg = _thresh(neg, rkn, MAX_NEG)
    sel_pos = jnp.logical_and(pos, rkp < t_pos)
    sel_neg = jnp.logical_and(neg, rkn < t_neg)

    zero = jnp.zeros_like(tx_ref[0])
    reg_ref[0, 0] = jnp.where(sel_pos, tx_ref[0], zero)
    reg_ref[0, 1] = jnp.where(sel_pos, ty_ref[0], zero)
    reg_ref[0, 2] = jnp.where(sel_pos, tw_ref[0], zero)
    reg_ref[0, 3] = jnp.where(sel_pos, th_ref[0], zero)
    reg_ref[0, 4] = pos.astype(jnp.float32)
    cls_ref[0, 0] = sel_neg.astype(jnp.float32)
    cls_ref[0, 1] = jnp.logical_or(pos, sel_neg).astype(jnp.float32)


def _forward_impl(anchors, batch_gt_boxes, batch_labels, interpret=False):
    rkp_np, rkn_np = _rank_tables()

    pad = jnp.broadcast_to(anchors[0], (AP - A, 4))
    aT = jnp.concatenate([anchors, pad], 0).T.reshape(4, ROWS, 128)
    gtT = jnp.transpose(batch_gt_boxes, (0, 2, 1))

    plane = jax.ShapeDtypeStruct((B, ROWS, 128), jnp.float32)
    mask_plane = jax.ShapeDtypeStruct((B, ROWS, 128), jnp.int32)
    blk_spec = lambda: pl.BlockSpec((1, BLK, 128), lambda b, a: (b, a, 0))

    pos, neg, tx, ty, tw, th = pl.pallas_call(
        _iou_kernel,
        grid=(B, ROWS // BLK),
        in_specs=[
            pl.BlockSpec((4, BLK, 128), lambda b, a: (0, a, 0)),
            pl.BlockSpec((1, 4, G), lambda b, a: (b, 0, 0),
                         memory_space=pltpu.SMEM),
            pl.BlockSpec((1, G), lambda b, a: (b, 0),
                         memory_space=pltpu.SMEM),
        ],
        out_specs=[blk_spec() for _ in range(6)],
        out_shape=[mask_plane, mask_plane, plane, plane, plane, plane],
        interpret=interpret,
    )(aT, gtT, batch_labels)

    full = lambda: pl.BlockSpec((1, ROWS, 128), lambda b: (b, 0, 0))
    reg_t, cls_t = pl.pallas_call(
        _select_kernel,
        grid=(B,),
        in_specs=[full() for _ in range(8)],
        out_specs=[
            pl.BlockSpec((1, 5, ROWS, 128), lambda b: (b, 0, 0, 0)),
            pl.BlockSpec((1, 2, ROWS, 128), lambda b: (b, 0, 0, 0)),
        ],
        out_shape=[
            jax.ShapeDtypeStruct((B, 5, ROWS, 128), jnp.float32),
            jax.ShapeDtypeStruct((B, 2, ROWS, 128), jnp.float32),
        ],
        interpret=interpret,
    )(pos, neg, jnp.asarray(rkp_np), jnp.asarray(rkn_np), tx, ty, tw, th)

    reg = reg_t.reshape(B, 5, AP).transpose(0, 2, 1)[:, :A, :]
    cls = cls_t.reshape(B, 2, AP).transpose(0, 2, 1)[:, :A, :]
    return reg, cls


def kernel(anchors, batch_gt_boxes, batch_labels):
    return _forward_impl(anchors, batch_gt_boxes, batch_labels)


# speedup vs baseline: 10.5424x; 10.5424x over previous
import numpy as np
import jax
import jax.numpy as jnp
from jax.experimental import pallas as pl
from jax.experimental.pallas import tpu as pltpu

POS_T = 0.7
NEG_T = 0.3
MAX_POS = 128
MAX_NEG = 256
A = 20000
B = 8
G = 64
AP = 20480
ROWS = AP // 128
PAD_RANK = 1 << 20

_rank_cache = []


def _rank_tables():
    if not _rank_cache:
        cpu = jax.devices("cpu")[0]
        rkp = np.full((B, AP), PAD_RANK, np.int32)
        rkn = np.full((B, AP), PAD_RANK, np.int32)
        with jax.ensure_compile_time_eval(), jax.default_device(cpu):
            key = jax.random.key(42)
            for b in range(B):
                kp, kn = jax.random.split(jax.random.fold_in(key, b))
                for arr, kk in ((rkp, kp), (rkn, kn)):
                    r = np.asarray(jax.random.uniform(kk, (A,)))
                    order = np.argsort(-r, kind="stable")
                    rank = np.empty(A, np.int32)
                    rank[order] = np.arange(A, dtype=np.int32)
                    arr[b, :A] = rank
        _rank_cache.append((rkp.reshape(B, ROWS, 128), rkn.reshape(B, ROWS, 128)))
    return _rank_cache[0]


BLK = 16


def _iou_kernel(aT_ref, gt_ref, lab_ref, pos_ref, neg_ref,
                tx_ref, ty_ref, tw_ref, th_ref):
    ax1 = aT_ref[0]
    ay1 = aT_ref[1]
    ax2 = aT_ref[2]
    ay2 = aT_ref[3]
    aw = ax2 - ax1
    ah = ay2 - ay1
    acx = (ax1 + ax2) / 2.0
    acy = (ay1 + ay2) / 2.0
    area1 = jnp.maximum(aw, 0.0) * jnp.maximum(ah, 0.0)

    shape = ax1.shape
    best_i = jnp.full(shape, -1.0, jnp.float32)
    best_u = jnp.full(shape, 1.0, jnp.float32)
    bbx = jnp.zeros(shape, jnp.float32)
    bby = jnp.zeros(shape, jnp.float32)
    bbw = jnp.ones(shape, jnp.float32)
    bbh = jnp.ones(shape, jnp.float32)
    pos_any = jnp.zeros(shape, jnp.bool_)
    negv_any = jnp.zeros(shape, jnp.bool_)

    for j in range(G):
        gx1 = gt_ref[0, 0, j]
        gy1 = gt_ref[0, 1, j]
        gx2 = gt_ref[0, 2, j]
        gy2 = gt_ref[0, 3, j]
        valid = lab_ref[0, 0, j] >= 0
        area2 = jnp.maximum(gx2 - gx1, 0.0) * jnp.maximum(gy2 - gy1, 0.0)

        ix1 = jnp.maximum(ax1, gx1)
        iy1 = jnp.maximum(ay1, gy1)
        ix2 = jnp.minimum(ax2, gx2)
        iy2 = jnp.minimum(ay2, gy2)
        iw = jnp.maximum(ix2 - ix1, 0.0)
        ih = jnp.maximum(iy2 - iy1, 0.0)
        inter = iw * ih
        s = area1 + area2
        pos_any = jnp.logical_or(pos_any,
                                 jnp.logical_and(17.0 * inter >= 7.0 * s, valid))
        negv_any = jnp.logical_or(negv_any,
                                  jnp.logical_and(13.0 * inter > 3.0 * s, valid))
        u = s - inter
        better = jnp.logical_and(inter * best_u > best_i * u, valid)
        best_i = jnp.where(better, inter, best_i)
        best_u = jnp.where(better, u, best_u)
        bbx = jnp.where(better, (gx1 + gx2) / 2.0, bbx)
        bby = jnp.where(better, (gy1 + gy2) / 2.0, bby)
        bbw = jnp.where(better, gx2 - gx1, bbw)
        bbh = jnp.where(better, gy2 - gy1, bbh)

    blk = pl.program_id(1)
    row = jax.lax.broadcasted_iota(jnp.int32, shape, 0)
    col = jax.lax.broadcasted_iota(jnp.int32, shape, 1)
    idx = (blk * BLK + row) * 128 + col
    in_range = idx < A
    pos_ref[0] = jnp.logical_and(pos_any, in_range).astype(jnp.int32)
    neg_ref[0] = jnp.logical_and(jnp.logical_not(negv_any), in_range).astype(jnp.int32)
    tx_ref[0] = (bbx - acx) / aw
    ty_ref[0] = (bby - acy) / ah
    tw_ref[0] = jnp.log(bbw / aw)
    th_ref[0] = jnp.log(bbh / ah)


def _thresh(mask, rank, k):
    t = jnp.int32(0)
    for b in range(14, -1, -1):
        cand = t + (1 << b)
        cnt = jnp.sum(jnp.where(jnp.logical_and(mask, rank < cand), 1, 0))
        t = jnp.where(cnt < k, cand, t)
    return t + 1


def _select_kernel(pos_ref, neg_ref, rkp_ref, rkn_ref,
                   tx_ref, ty_ref, tw_ref, th_ref, reg_ref, cls_ref):
    pos = pos_ref[0] != 0
    neg = neg_ref[0] != 0
    rkp = rkp_ref[0]
    rkn = rkn_ref[0]
    t_pos = _thresh(pos, rkp, MAX_POS)
    t_neg = _thresh(neg, rkn, MAX_NEG)
    sel_pos = jnp.logical_and(pos, rkp < t_pos)
    sel_neg = jnp.logical_and(neg, rkn < t_neg)

    zero = jnp.zeros_like(tx_ref[0])
    reg_ref[0, 0] = jnp.where(sel_pos, tx_ref[0], zero)
    reg_ref[0, 1] = jnp.where(sel_pos, ty_ref[0], zero)
    reg_ref[0, 2] = jnp.where(sel_pos, tw_ref[0], zero)
    reg_ref[0, 3] = jnp.where(sel_pos, th_ref[0], zero)
    reg_ref[0, 4] = pos.astype(jnp.float32)
    cls_ref[0, 0] = sel_neg.astype(jnp.float32)
    cls_ref[0, 1] = jnp.logical_or(pos, sel_neg).astype(jnp.float32)


def _forward_impl(anchors, batch_gt_boxes, batch_labels, interpret=False):
    rkp_np, rkn_np = _rank_tables()

    pad = jnp.broadcast_to(anchors[0], (AP - A, 4))
    aT = jnp.concatenate([anchors, pad], 0).T.reshape(4, ROWS, 128)
    gtT = jnp.transpose(batch_gt_boxes, (0, 2, 1))

    plane = jax.ShapeDtypeStruct((B, ROWS, 128), jnp.float32)
    mask_plane = jax.ShapeDtypeStruct((B, ROWS, 128), jnp.int32)
    blk_spec = lambda: pl.BlockSpec((1, BLK, 128), lambda b, a: (b, a, 0))

    pos, neg, tx, ty, tw, th = pl.pallas_call(
        _iou_kernel,
        grid=(B, ROWS // BLK),
        in_specs=[
            pl.BlockSpec((4, BLK, 128), lambda b, a: (0, a, 0)),
            pl.BlockSpec((1, 4, G), lambda b, a: (b, 0, 0),
                         memory_space=pltpu.SMEM),
            pl.BlockSpec((1, 1, G), lambda b, a: (b, 0, 0),
                         memory_space=pltpu.SMEM),
        ],
        out_specs=[blk_spec() for _ in range(6)],
        out_shape=[mask_plane, mask_plane, plane, plane, plane, plane],
        interpret=interpret,
    )(aT, gtT, batch_labels.reshape(B, 1, G))

    full = lambda: pl.BlockSpec((1, ROWS, 128), lambda b: (b, 0, 0))
    reg_t, cls_t = pl.pallas_call(
        _select_kernel,
        grid=(B,),
        in_specs=[full() for _ in range(8)],
        out_specs=[
            pl.BlockSpec((1, 5, ROWS, 128), lambda b: (b, 0, 0, 0)),
            pl.BlockSpec((1, 2, ROWS, 128), lambda b: (b, 0, 0, 0)),
        ],
        out_shape=[
            jax.ShapeDtypeStruct((B, 5, ROWS, 128), jnp.float32),
            jax.ShapeDtypeStruct((B, 2, ROWS, 128), jnp.float32),
        ],
        interpret=interpret,
    )(pos, neg, jnp.asarray(rkp_np), jnp.asarray(rkn_np), tx, ty, tw, th)

    reg = reg_t.reshape(B, 5, AP).transpose(0, 2, 1)[:, :A, :]
    cls = cls_t.reshape(B, 2, AP).transpose(0, 2, 1)[:, :A, :]
    return reg, cls


def kernel(anchors, batch_gt_boxes, batch_labels):
    return _forward_impl(anchors, batch_gt_boxes, batch_labels)
